# baseline (speedup 1.0000x reference)
import jax
import jax.numpy as jnp
from jax import lax
from jax.experimental import pallas as pl
from jax.experimental.pallas import tpu as pltpu
from jax.experimental.pallas import tpu_sc as plsc

N = 320000
D = 128
S = 10000
EPS = 1e-5

P = 4
NP = N // P
ROW_BLOCK = 16000
CHUNK = 128
NC = 2
NS = 16
NW = NC * NS
SP = 10240
ROWS_PER_SUB = SP // NS

BLK = 128
NBLK_P = NP // BLK
BASE_BLK = NBLK_P // NW
EXTRA = NBLK_P - BASE_BLK * NW
T_OUTER = (BASE_BLK + 2) // 2


def _linear_ln_body(x_ref, wt_ref, b_ref, g_ref, bt_ref, h_ref):
    h = jnp.dot(x_ref[...], wt_ref[...], preferred_element_type=jnp.float32)
    h = h + b_ref[...]
    mu = jnp.mean(h, axis=-1, keepdims=True)
    var = jnp.mean((h - mu) ** 2, axis=-1, keepdims=True)
    h_ref[...] = (h - mu) * lax.rsqrt(var + EPS) * g_ref[...] + bt_ref[...]


def _linear_ln(x, wt, b2, g2, bt2, piece):
    grid = (NP // ROW_BLOCK,)
    off = piece * (NP // ROW_BLOCK)
    return pl.pallas_call(
        _linear_ln_body,
        grid=grid,
        in_specs=[
            pl.BlockSpec((ROW_BLOCK, D), lambda i: (i + off, 0)),
            pl.BlockSpec((D, D), lambda i: (0, 0)),
            pl.BlockSpec((1, D), lambda i: (0, 0)),
            pl.BlockSpec((1, D), lambda i: (0, 0)),
            pl.BlockSpec((1, D), lambda i: (0, 0)),
        ],
        out_specs=pl.BlockSpec((ROW_BLOCK, D), lambda i: (i, 0)),
        out_shape=jax.ShapeDtypeStruct((NP, D), jnp.float32),
    )(x, wt, b2, g2, bt2)


def _make_sc_body(blk0):
    def _sc_body(h_hbm, b2d_hbm, zrow_hbm, zcnt_hbm, ones_hbm,
                 psum_hbm, cnt_hbm,
                 acc, cacc, idx_v, rows_v, ones_v, zc16_v,
                 lsem0, lsem1, ssem0, ssem1):
        cid = lax.axis_index("c")
        sid = lax.axis_index("s")
        wid = cid * NS + sid
        base = sid * ROWS_PER_SUB
        lsem = (lsem0, lsem1)
        ssem = (ssem0, ssem1)

        start = BASE_BLK * wid + jnp.minimum(wid, EXTRA)
        nblk = BASE_BLK + jnp.where(wid < EXTRA, 1, 0)

        pltpu.sync_copy(zrow_hbm, rows_v.at[0])
        pltpu.sync_copy(zcnt_hbm, zc16_v)
        pltpu.sync_copy(ones_hbm, ones_v)
        for j in range(ROWS_PER_SUB // CHUNK):
            pltpu.sync_copy(rows_v.at[0],
                            acc.at[pl.ds(base + j * CHUNK, CHUNK)])
        for j in range(ROWS_PER_SUB // CHUNK):
            pltpu.sync_copy(zc16_v,
                            cacc.at[pl.ds(base + j * CHUNK, CHUNK)])
        plsc.subcore_barrier()

        def issue_load(blk, buf):
            pltpu.async_copy(b2d_hbm.at[pl.ds((blk0 + blk) * CHUNK, CHUNK)],
                             idx_v.at[buf, 0], lsem[buf])
            pltpu.async_copy(h_hbm.at[pl.ds(blk * BLK, BLK)], rows_v.at[buf],
                             lsem[buf])

        def wait_load(blk, buf):
            pltpu.make_async_copy(
                b2d_hbm.at[pl.ds((blk0 + blk) * CHUNK, CHUNK)],
                idx_v.at[buf, 0], lsem[buf]).wait()
            pltpu.make_async_copy(
                h_hbm.at[pl.ds(blk * BLK, BLK)], rows_v.at[buf],
                lsem[buf]).wait()

        def issue_scat(buf):
            pltpu.async_copy(rows_v.at[buf], acc.at[idx_v.at[buf, 0]],
                             ssem[buf], add=True)
            pltpu.async_copy(ones_v, cacc.at[idx_v.at[buf, 0]], ssem[buf],
                             add=True)

        def wait_scat(buf):
            pltpu.make_async_copy(rows_v.at[buf], acc.at[idx_v.at[buf, 0]],
                                  ssem[buf]).wait()
            pltpu.make_async_copy(ones_v, cacc.at[idx_v.at[buf, 0]],
                                  ssem[buf]).wait()

        issue_load(start, 0)

        def t_body(t, carry):
            for half in range(2):
                k = 2 * t + half
                buf = half

                @pl.when(k < nblk)
                def _():
                    wait_load(start + k, buf)
                    issue_scat(buf)

                @pl.when(k + 1 < nblk)
                def _():
                    @pl.when(k >= 1)
                    def __():
                        wait_scat(1 - buf)

                    issue_load(start + k + 1, 1 - buf)

            return carry

        lax.fori_loop(0, T_OUTER, t_body, 0)
        wait_scat(0)
        wait_scat(1)
        plsc.subcore_barrier()

        for j in range(ROWS_PER_SUB // CHUNK):
            pltpu.sync_copy(acc.at[pl.ds(base + j * CHUNK, CHUNK)],
                            rows_v.at[j % 2])
            pltpu.sync_copy(rows_v.at[j % 2],
                            psum_hbm.at[cid, pl.ds(base + j * CHUNK, CHUNK)])
        for j in range(ROWS_PER_SUB // CHUNK):
            pltpu.sync_copy(cacc.at[pl.ds(base + j * CHUNK, CHUNK)], zc16_v)
            pltpu.sync_copy(zc16_v,
                            cnt_hbm.at[cid, pl.ds(base + j * CHUNK, CHUNK)])

    return _sc_body


def _segment_sums(h, batch, zrow, zcnt, ones, piece):
    mesh = plsc.VectorSubcoreMesh(core_axis_name="c", subcore_axis_name="s")
    return pl.kernel(
        _make_sc_body(piece * NBLK_P),
        out_type=[
            jax.ShapeDtypeStruct((NC, SP, D), jnp.float32),
            jax.ShapeDtypeStruct((NC, SP, 16), jnp.float32),
        ],
        mesh=mesh,
        compiler_params=pltpu.CompilerParams(use_tc_tiling_on_sc=False),
        scratch_types=[
            pltpu.VMEM_SHARED((SP, D), jnp.float32),
            pltpu.VMEM_SHARED((SP, 16), jnp.float32),
            pltpu.VMEM((2, 1, CHUNK), jnp.int32),
            pltpu.VMEM((2, BLK, D), jnp.float32),
            pltpu.VMEM((CHUNK, 16), jnp.float32),
            pltpu.VMEM((CHUNK, 16), jnp.float32),
            pltpu.SemaphoreType.DMA,
            pltpu.SemaphoreType.DMA,
            pltpu.SemaphoreType.DMA,
            pltpu.SemaphoreType.DMA,
        ],
    )(h, batch, zrow, zcnt, ones)


OUT_BLK = 2000


def _combine_body(*refs):
    p_refs = refs[:P]
    c_refs = refs[P:2 * P]
    o_ref = refs[2 * P]
    cnt = sum(c[i, :, 0:1] for c in c_refs for i in range(NC))
    tot = sum(p[i] for p in p_refs for i in range(NC))
    o_ref[...] = tot / jnp.maximum(cnt, 1.0)


def _combine(psums, cnts):
    grid = (S // OUT_BLK,)
    pspec = pl.BlockSpec((NC, OUT_BLK, D), lambda i: (0, i, 0))
    cspec = pl.BlockSpec((NC, OUT_BLK, 16), lambda i: (0, i, 0))
    return pl.pallas_call(
        _combine_body,
        grid=grid,
        in_specs=[pspec] * P + [cspec] * P,
        out_specs=pl.BlockSpec((OUT_BLK, D), lambda i: (i, 0)),
        out_shape=jax.ShapeDtypeStruct((S, D), jnp.float32),
    )(*psums, *cnts)


def kernel(x, batch, W, b, gamma, beta):
    wt = W.T
    b2 = b.reshape(1, D)
    g2 = gamma.reshape(1, D)
    bt2 = beta.reshape(1, D)
    zrow = jnp.zeros((BLK, D), jnp.float32)
    zcnt = jnp.zeros((CHUNK, 16), jnp.float32)
    ones = jnp.ones((CHUNK, 16), jnp.float32)
    psums, cnts = [], []
    for piece in range(P):
        h = _linear_ln(x, wt, b2, g2, bt2, piece)
        ps, cn = _segment_sums(h, batch, zrow, zcnt, ones, piece)
        psums.append(ps)
        cnts.append(cn)
    return _combine(psums, cnts)


# speedup vs baseline: 1.2534x; 1.2534x over previous
import jax
import jax.numpy as jnp
from jax import lax
from jax.experimental import pallas as pl
from jax.experimental.pallas import tpu as pltpu
from jax.experimental.pallas import tpu_sc as plsc

N = 320000
D = 128
S = 10000
EPS = 1e-5

P = 2
NP = N // P
ROW_BLOCK = 16000
CHUNK = 128
NC = 2
NS = 16
NW = NC * NS
SP = 10240
ROWS_PER_SUB = SP // NS

BLK = 128
NBLK_P = NP // BLK
BASE_BLK = NBLK_P // NW
EXTRA = NBLK_P - BASE_BLK * NW
T_OUTER = (BASE_BLK + 2) // 2

IDB = 1280
NBI = N // IDB
BASE_I = NBI // NW
EXTRA_I = NBI - BASE_I * NW
T_OUTER_I = (BASE_I + 2) // 2


def _linear_ln_body(x_ref, wt_ref, b_ref, g_ref, bt_ref, h_ref):
    h = jnp.dot(x_ref[...], wt_ref[...], preferred_element_type=jnp.float32)
    h = h + b_ref[...]
    mu = jnp.mean(h, axis=-1, keepdims=True)
    var = jnp.mean((h - mu) ** 2, axis=-1, keepdims=True)
    h_ref[...] = (h - mu) * lax.rsqrt(var + EPS) * g_ref[...] + bt_ref[...]


def _linear_ln(x, wt, b2, g2, bt2, piece):
    grid = (NP // ROW_BLOCK,)
    off = piece * (NP // ROW_BLOCK)
    return pl.pallas_call(
        _linear_ln_body,
        grid=grid,
        in_specs=[
            pl.BlockSpec((ROW_BLOCK, D), lambda i: (i + off, 0)),
            pl.BlockSpec((D, D), lambda i: (0, 0)),
            pl.BlockSpec((1, D), lambda i: (0, 0)),
            pl.BlockSpec((1, D), lambda i: (0, 0)),
            pl.BlockSpec((1, D), lambda i: (0, 0)),
        ],
        out_specs=pl.BlockSpec((ROW_BLOCK, D), lambda i: (i, 0)),
        out_shape=jax.ShapeDtypeStruct((NP, D), jnp.float32),
    )(x, wt, b2, g2, bt2)


def _cnt_body(b_hbm, zcnt_hbm, ones_hbm, cnt_hbm,
              cacc, idx_v, ones_v, zc16_v, lsem0, lsem1, ssem0, ssem1):
    cid = lax.axis_index("c")
    sid = lax.axis_index("s")
    wid = cid * NS + sid
    base = sid * ROWS_PER_SUB
    lsem = (lsem0, lsem1)
    ssem = (ssem0, ssem1)

    start = BASE_I * wid + jnp.minimum(wid, EXTRA_I)
    nblk = BASE_I + jnp.where(wid < EXTRA_I, 1, 0)

    pltpu.sync_copy(zcnt_hbm, zc16_v)
    pltpu.sync_copy(ones_hbm, ones_v)
    for j in range(ROWS_PER_SUB // CHUNK):
        pltpu.sync_copy(zc16_v, cacc.at[pl.ds(base + j * CHUNK, CHUNK)])
    plsc.subcore_barrier()

    def issue_load(blk, buf):
        pltpu.async_copy(b_hbm.at[blk], idx_v.at[buf], lsem[buf])

    def wait_load(blk, buf):
        pltpu.make_async_copy(b_hbm.at[blk], idx_v.at[buf], lsem[buf]).wait()

    def issue_scat(buf):
        for j in range(IDB // CHUNK):
            pltpu.async_copy(ones_v, cacc.at[idx_v.at[buf, j]], ssem[buf],
                             add=True)

    def wait_scat(buf):
        for j in range(IDB // CHUNK):
            pltpu.make_async_copy(ones_v, cacc.at[idx_v.at[buf, j]],
                                  ssem[buf]).wait()

    issue_load(start, 0)

    def t_body(t, carry):
        for half in range(2):
            k = 2 * t + half
            buf = half

            @pl.when(k < nblk)
            def _():
                wait_load(start + k, buf)
                issue_scat(buf)

            @pl.when(k + 1 < nblk)
            def _():
                @pl.when(k >= 1)
                def __():
                    wait_scat(1 - buf)

                issue_load(start + k + 1, 1 - buf)

        return carry

    lax.fori_loop(0, T_OUTER_I, t_body, 0)
    wait_scat(0)
    wait_scat(1)
    plsc.subcore_barrier()

    for j in range(ROWS_PER_SUB // CHUNK):
        pltpu.sync_copy(cacc.at[pl.ds(base + j * CHUNK, CHUNK)], zc16_v)
        pltpu.sync_copy(zc16_v, cnt_hbm.at[cid, pl.ds(base + j * CHUNK, CHUNK)])


def _segment_counts(batch, zcnt, ones):
    mesh = plsc.VectorSubcoreMesh(core_axis_name="c", subcore_axis_name="s")
    return pl.kernel(
        _cnt_body,
        out_type=jax.ShapeDtypeStruct((NC, SP, 16), jnp.float32),
        mesh=mesh,
        compiler_params=pltpu.CompilerParams(use_tc_tiling_on_sc=False),
        scratch_types=[
            pltpu.VMEM_SHARED((SP, 16), jnp.float32),
            pltpu.VMEM((2, IDB // CHUNK, CHUNK), jnp.int32),
            pltpu.VMEM((CHUNK, 16), jnp.float32),
            pltpu.VMEM((CHUNK, 16), jnp.float32),
            pltpu.SemaphoreType.DMA,
            pltpu.SemaphoreType.DMA,
            pltpu.SemaphoreType.DMA,
            pltpu.SemaphoreType.DMA,
        ],
    )(batch, zcnt, ones)


def _make_sc_body(blk0):
    def _sc_body(h_hbm, b2d_hbm, zrow_hbm, psum_hbm,
                 acc, idx_v, rows_v, lsem0, lsem1, ssem0, ssem1):
        cid = lax.axis_index("c")
        sid = lax.axis_index("s")
        wid = cid * NS + sid
        base = sid * ROWS_PER_SUB
        lsem = (lsem0, lsem1)
        ssem = (ssem0, ssem1)

        start = BASE_BLK * wid + jnp.minimum(wid, EXTRA)
        nblk = BASE_BLK + jnp.where(wid < EXTRA, 1, 0)

        pltpu.sync_copy(zrow_hbm, rows_v.at[0])
        for j in range(ROWS_PER_SUB // CHUNK):
            pltpu.sync_copy(rows_v.at[0],
                            acc.at[pl.ds(base + j * CHUNK, CHUNK)])
        plsc.subcore_barrier()

        def issue_load(blk, buf):
            pltpu.async_copy(b2d_hbm.at[pl.ds((blk0 + blk) * CHUNK, CHUNK)],
                             idx_v.at[buf, 0], lsem[buf])
            pltpu.async_copy(h_hbm.at[pl.ds(blk * BLK, BLK)], rows_v.at[buf],
                             lsem[buf])

        def wait_load(blk, buf):
            pltpu.make_async_copy(
                b2d_hbm.at[pl.ds((blk0 + blk) * CHUNK, CHUNK)],
                idx_v.at[buf, 0], lsem[buf]).wait()
            pltpu.make_async_copy(
                h_hbm.at[pl.ds(blk * BLK, BLK)], rows_v.at[buf],
                lsem[buf]).wait()

        def issue_scat(buf):
            pltpu.async_copy(rows_v.at[buf], acc.at[idx_v.at[buf, 0]],
                             ssem[buf], add=True)

        def wait_scat(buf):
            pltpu.make_async_copy(rows_v.at[buf], acc.at[idx_v.at[buf, 0]],
                                  ssem[buf]).wait()

        issue_load(start, 0)

        def t_body(t, carry):
            for half in range(2):
                k = 2 * t + half
                buf = half

                @pl.when(k < nblk)
                def _():
                    wait_load(start + k, buf)
                    issue_scat(buf)

                @pl.when(k + 1 < nblk)
                def _():
                    @pl.when(k >= 1)
                    def __():
                        wait_scat(1 - buf)

                    issue_load(start + k + 1, 1 - buf)

            return carry

        lax.fori_loop(0, T_OUTER, t_body, 0)
        wait_scat(0)
        wait_scat(1)
        plsc.subcore_barrier()

        for j in range(ROWS_PER_SUB // CHUNK):
            pltpu.sync_copy(acc.at[pl.ds(base + j * CHUNK, CHUNK)],
                            rows_v.at[j % 2])
            pltpu.sync_copy(rows_v.at[j % 2],
                            psum_hbm.at[cid, pl.ds(base + j * CHUNK, CHUNK)])

    return _sc_body


def _segment_sums(h, batch, zrow, piece):
    mesh = plsc.VectorSubcoreMesh(core_axis_name="c", subcore_axis_name="s")
    return pl.kernel(
        _make_sc_body(piece * NBLK_P),
        out_type=jax.ShapeDtypeStruct((NC, SP, D), jnp.float32),
        mesh=mesh,
        compiler_params=pltpu.CompilerParams(use_tc_tiling_on_sc=False),
        scratch_types=[
            pltpu.VMEM_SHARED((SP, D), jnp.float32),
            pltpu.VMEM((2, 1, CHUNK), jnp.int32),
            pltpu.VMEM((2, BLK, D), jnp.float32),
            pltpu.SemaphoreType.DMA,
            pltpu.SemaphoreType.DMA,
            pltpu.SemaphoreType.DMA,
            pltpu.SemaphoreType.DMA,
        ],
    )(h, batch, zrow)


OUT_BLK = 2000


def _combine_body(*refs):
    p_refs = refs[:P]
    c_ref = refs[P]
    o_ref = refs[P + 1]
    cnt = c_ref[0, :, 0:1] + c_ref[1, :, 0:1]
    tot = sum(p[i] for p in p_refs for i in range(NC))
    o_ref[...] = tot / jnp.maximum(cnt, 1.0)


def _combine(psums, cnt):
    grid = (S // OUT_BLK,)
    pspec = pl.BlockSpec((NC, OUT_BLK, D), lambda i: (0, i, 0))
    cspec = pl.BlockSpec((NC, OUT_BLK, 16), lambda i: (0, i, 0))
    return pl.pallas_call(
        _combine_body,
        grid=grid,
        in_specs=[pspec] * P + [cspec],
        out_specs=pl.BlockSpec((OUT_BLK, D), lambda i: (i, 0)),
        out_shape=jax.ShapeDtypeStruct((S, D), jnp.float32),
    )(*psums, cnt)


def kernel(x, batch, W, b, gamma, beta):
    wt = W.T
    b2 = b.reshape(1, D)
    g2 = gamma.reshape(1, D)
    bt2 = beta.reshape(1, D)
    zrow = jnp.zeros((BLK, D), jnp.float32)
    zcnt = jnp.zeros((CHUNK, 16), jnp.float32)
    ones = jnp.ones((CHUNK, 16), jnp.float32)
    cnt = _segment_counts(batch.reshape(NBI, IDB // CHUNK, CHUNK), zcnt, ones)
    psums = []
    for piece in range(P):
        h = _linear_ln(x, wt, b2, g2, bt2, piece)
        psums.append(_segment_sums(h, batch, zrow, piece))
    return _combine(psums, cnt)
